# Initial kernel scaffold; baseline (speedup 1.0000x reference)
import jax
import jax.numpy as jnp
from jax.experimental import pallas as pl


def kernel(Y, edge_index, edge_weight, W1, b1, W_mu, b_mu, W_lv, b_lv):
    raise NotImplementedError("write your pallas kernel here")


# speedup vs baseline: 10.6989x; 10.6989x over previous
import functools

import jax
import jax.numpy as jnp
from jax import lax
from jax.experimental import pallas as pl
from jax.experimental.pallas import tpu as pltpu
from jax.experimental.pallas import tpu_sc as plsc

N = 10000
E = 320000
D = 128
NC, NS = 2, 16
NW = NC * NS
EP = E // NW
CH = 80
NCHUNK = EP // CH
NPAD = 10240
RPT = NPAD // NS

_mesh = plsc.VectorSubcoreMesh(core_axis_name="c", subcore_axis_name="s")


def _deg_body(dst_hbm, w_hbm, out_hbm, dstv, wv, zv, degsp):
    c = lax.axis_index("c")
    s = lax.axis_index("s")
    wid = s * NC + c
    for i in range(RPT // 16):
        zv[pl.ds(i * 16, 16)] = jnp.zeros((16,), jnp.float32)
    pltpu.sync_copy(zv, degsp.at[pl.ds(s * RPT, RPT)])
    plsc.subcore_barrier()

    def chunk(g, _):
        base = wid * EP + g * CH
        pltpu.sync_copy(dst_hbm.at[pl.ds(base, CH)], dstv)
        pltpu.sync_copy(w_hbm.at[pl.ds(base, CH)], wv)
        pltpu.sync_copy(wv, degsp.at[dstv], add=True)
        return 0

    lax.fori_loop(0, NCHUNK, chunk, 0)
    plsc.subcore_barrier()
    pltpu.sync_copy(degsp.at[pl.ds(s * RPT, RPT)],
                    out_hbm.at[pl.ds(c * NPAD + s * RPT, RPT)])


_deg_call = functools.partial(
    pl.kernel,
    out_type=jax.ShapeDtypeStruct((NC * NPAD,), jnp.float32),
    mesh=_mesh,
    scratch_types=[
        pltpu.VMEM((CH,), jnp.int32),
        pltpu.VMEM((CH,), jnp.float32),
        pltpu.VMEM((RPT,), jnp.float32),
        pltpu.VMEM_SHARED((NPAD,), jnp.float32),
    ],
)(_deg_body)


def _prop_body(xs_hbm, half_hbm, src_hbm, dst_hbm, w_hbm, out_hbm,
               srcv, dstv, wv, rows, sem, accsp):
    c = lax.axis_index("c")
    s = lax.axis_index("s")
    wid = s * NC + c
    r0 = s * RPT
    pltpu.sync_copy(half_hbm.at[pl.ds(r0, RPT)], accsp.at[pl.ds(r0, RPT)])
    plsc.subcore_barrier()

    def chunk(g, _):
        base = wid * EP + g * CH
        pltpu.sync_copy(src_hbm.at[pl.ds(base, CH)], srcv)
        pltpu.sync_copy(dst_hbm.at[pl.ds(base, CH)], dstv)
        pltpu.sync_copy(w_hbm.at[pl.ds(base, CH)], wv.at[pl.ds(0, CH)])
        pltpu.async_copy(xs_hbm.at[srcv], rows, sem).wait()

        def scale(j, _):
            wj = wv[pl.ds(j, 16)][0]
            for k in range(D // 16):
                sl = pl.ds(k * 16, 16)
                rows[j, sl] = rows[j, sl] * wj
            return 0

        lax.fori_loop(0, CH, scale, 0)
        pltpu.sync_copy(rows, accsp.at[dstv], add=True)
        return 0

    lax.fori_loop(0, NCHUNK, chunk, 0)
    plsc.subcore_barrier()
    pltpu.sync_copy(accsp.at[pl.ds(r0, RPT)],
                    out_hbm.at[pl.ds(c * NPAD + r0, RPT)])


_prop_call = functools.partial(
    pl.kernel,
    out_type=jax.ShapeDtypeStruct((NC * NPAD, D), jnp.float32),
    mesh=_mesh,
    scratch_types=[
        pltpu.VMEM((CH,), jnp.int32),
        pltpu.VMEM((CH,), jnp.int32),
        pltpu.VMEM((CH + 16,), jnp.float32),
        pltpu.VMEM((CH, D), jnp.float32),
        pltpu.SemaphoreType.DMA,
        pltpu.VMEM_SHARED((NPAD, D), jnp.float32),
    ],
)(_prop_body)


def _tc1_body(y_ref, w1_ref, degp_ref, xs_ref, half_ref, dinv_ref):
    xw = jnp.dot(y_ref[...], w1_ref[...], preferred_element_type=jnp.float32)
    deg = degp_ref[0, :N, :] + degp_ref[1, :N, :] + 1.0
    dinv = jnp.where(deg > 0, lax.rsqrt(deg), 0.0)
    xs = xw * dinv
    xs_ref[:N, :] = xs
    xs_ref[pl.ds(N, NPAD - N), :] = jnp.zeros((NPAD - N, D), jnp.float32)
    half_ref[:N, :] = 0.5 * xs
    half_ref[pl.ds(N, NPAD - N), :] = jnp.zeros((NPAD - N, D), jnp.float32)
    dinv_ref[...] = dinv


def _tc1(Y, W1, degp3):
    return pl.pallas_call(
        _tc1_body,
        out_shape=[
            jax.ShapeDtypeStruct((NPAD, D), jnp.float32),
            jax.ShapeDtypeStruct((NPAD, D), jnp.float32),
            jax.ShapeDtypeStruct((N, 1), jnp.float32),
        ],
    )(Y, W1, degp3)


def _tc2_body(acc_ref, dinv_ref, b1_ref, hs_ref, half2_ref):
    s1 = acc_ref[:N, :] + acc_ref[pl.ds(NPAD, N), :]
    dinv = dinv_ref[...]
    h = jax.nn.relu(s1 * dinv + b1_ref[...])
    hs = h * dinv
    hs_ref[:N, :] = hs
    hs_ref[pl.ds(N, NPAD - N), :] = jnp.zeros((NPAD - N, D), jnp.float32)
    half2_ref[:N, :] = 0.5 * hs
    half2_ref[pl.ds(N, NPAD - N), :] = jnp.zeros((NPAD - N, D), jnp.float32)


def _tc2(acc, dinv, b1):
    return pl.pallas_call(
        _tc2_body,
        out_shape=[
            jax.ShapeDtypeStruct((NPAD, D), jnp.float32),
            jax.ShapeDtypeStruct((NPAD, D), jnp.float32),
        ],
    )(acc, dinv, b1)


def _tc3_body(acc_ref, dinv_ref, wmu_ref, bmu_ref, wlv_ref, blv_ref,
              mu_ref, lv_ref):
    p2 = (acc_ref[:N, :] + acc_ref[pl.ds(NPAD, N), :]) * dinv_ref[...]
    mu_ref[...] = jnp.dot(p2, wmu_ref[...],
                          preferred_element_type=jnp.float32) + bmu_ref[...]
    lv_ref[...] = jnp.dot(p2, wlv_ref[...],
                          preferred_element_type=jnp.float32) + blv_ref[...]


def _tc3(acc, dinv, W_mu, b_mu, W_lv, b_lv):
    lat = W_mu.shape[1]
    return pl.pallas_call(
        _tc3_body,
        out_shape=[
            jax.ShapeDtypeStruct((N, lat), jnp.float32),
            jax.ShapeDtypeStruct((N, lat), jnp.float32),
        ],
    )(acc, dinv, W_mu, b_mu, W_lv, b_lv)


@jax.jit
def kernel(Y, edge_index, edge_weight, W1, b1, W_mu, b_mu, W_lv, b_lv):
    src = edge_index[0].astype(jnp.int32)
    dst = edge_index[1].astype(jnp.int32)
    w = edge_weight.astype(jnp.float32)

    degp = _deg_call(dst, w)
    degp3 = degp.reshape(NC, NPAD, 1)
    Xs, half, dinv = _tc1(Y, W1, degp3)
    acc1 = _prop_call(Xs, half, src, dst, w)
    Hs, half2 = _tc2(acc1, dinv, b1)
    acc2 = _prop_call(Hs, half2, src, dst, w)
    mu, lv = _tc3(acc2, dinv, W_mu, b_mu, W_lv, b_lv)
    return (mu, lv)


# speedup vs baseline: 25.6736x; 2.3996x over previous
import functools

import jax
import jax.numpy as jnp
from jax import lax
from jax.experimental import pallas as pl
from jax.experimental.pallas import tpu as pltpu
from jax.experimental.pallas import tpu_sc as plsc

N = 10000
E = 320000
D = 128
HD = D // 2
NC, NS = 2, 16
NW = NC * NS
CH = 80
NB = 5
ECS = E // NS
NCHUNK = ECS // CH
EPD = E // NW
NCHUNK_D = EPD // CH
NPAD = 10240
RPT = NPAD // NS

_mesh = plsc.VectorSubcoreMesh(core_axis_name="c", subcore_axis_name="s")


def _deg_body(dst3_hbm, w3_hbm, out_hbm, dstall, wall, zv, dsem, degsp):
    c = lax.axis_index("c")
    s = lax.axis_index("s")
    wid = s * NC + c
    for i in range(RPT // 16):
        zv[pl.ds(i * 16, 16)] = jnp.zeros((16,), jnp.float32)
    pltpu.sync_copy(zv, degsp.at[pl.ds(s * RPT, RPT)])
    pltpu.sync_copy(dst3_hbm.at[wid], dstall)
    pltpu.sync_copy(w3_hbm.at[wid], wall)
    plsc.subcore_barrier()

    def fire(g, _):
        pltpu.async_copy(wall.at[g], degsp.at[dstall.at[g]], dsem, add=True)

        @pl.when(g >= 16)
        def _wait():
            pltpu.make_async_copy(wall.at[0], degsp.at[pl.ds(0, CH)],
                                  dsem).wait()
        return 0

    lax.fori_loop(0, NCHUNK_D, fire, 0)
    for _ in range(16):
        pltpu.make_async_copy(wall.at[0], degsp.at[pl.ds(0, CH)], dsem).wait()
    plsc.subcore_barrier()
    pltpu.sync_copy(degsp.at[pl.ds(s * RPT, RPT)],
                    out_hbm.at[pl.ds(c * NPAD + s * RPT, RPT)])


_deg_call = functools.partial(
    pl.kernel,
    out_type=jax.ShapeDtypeStruct((NC * NPAD,), jnp.float32),
    mesh=_mesh,
    scratch_types=[
        pltpu.VMEM((NCHUNK_D, CH), jnp.int32),
        pltpu.VMEM((NCHUNK_D, CH), jnp.float32),
        pltpu.VMEM((RPT,), jnp.float32),
        pltpu.SemaphoreType.DMA,
        pltpu.VMEM_SHARED((NPAD,), jnp.float32),
    ],
)(_deg_body)


def _prop_body(xs2_hbm, src_hbm, dst_hbm, w_hbm, out_hbm, *rest):
    sv = rest[0:NB]
    dv = rest[NB:2 * NB]
    wv = rest[2 * NB:3 * NB]
    rows = rest[3 * NB:4 * NB]
    isem = rest[4 * NB:5 * NB]
    gsem = rest[5 * NB:6 * NB]
    ssem = rest[6 * NB:7 * NB]
    accsp = rest[7 * NB]
    c = lax.axis_index("c")
    s = lax.axis_index("s")
    r0 = s * RPT
    e0 = s * ECS
    pltpu.sync_copy(xs2_hbm.at[c, pl.ds(r0, RPT)], accsp.at[pl.ds(r0, RPT)])

    def start_idx(g, b):
        base = e0 + g * CH
        pltpu.async_copy(src_hbm.at[pl.ds(base, CH)], sv[b], isem[b])
        pltpu.async_copy(dst_hbm.at[pl.ds(base, CH)], dv[b], isem[b])
        pltpu.async_copy(w_hbm.at[pl.ds(base, CH)], wv[b], isem[b])

    def wait_idx(b):
        for _ in range(3):
            pltpu.make_async_copy(src_hbm.at[pl.ds(0, CH)], sv[b],
                                  isem[b]).wait()

    def wait_rows_sem(sem, b):
        pltpu.make_async_copy(xs2_hbm.at[c, pl.ds(0, CH)], rows[b],
                              sem[b]).wait()

    start_idx(0, 0)
    start_idx(1, 1)
    plsc.subcore_barrier()
    wait_idx(0)
    pltpu.async_copy(xs2_hbm.at[c].at[sv[0]], rows[0], gsem[0])

    def scale(b, g):
        del g

        def sixteen(t, _):
            wrow = wv[b][pl.ds(t * 16, 16)]
            for l in range(16):
                wj = wrow[l]
                j = t * 16 + l
                for k in range(HD // 16):
                    sl = pl.ds(k * 16, 16)
                    rows[b][j, sl] = rows[b][j, sl] * wj
            return 0

        lax.fori_loop(0, CH // 16, sixteen, 0)

    def group(grp, _):
        for b in range(NB):
            g = grp * NB + b
            b2 = (b + 1) % NB
            b3 = (b + 2) % NB

            @pl.when(g + 2 < NCHUNK)
            def _prefetch_idx():
                @pl.when(g >= NB - 2)
                def _free():
                    wait_rows_sem(ssem, b3)
                start_idx(g + 2, b3)

            @pl.when(g + 1 < NCHUNK)
            def _prefetch_rows():
                wait_idx(b2)
                pltpu.async_copy(xs2_hbm.at[c].at[sv[b2]], rows[b2],
                                 gsem[b2])

            wait_rows_sem(gsem, b)
            scale(b, g)
            pltpu.async_copy(rows[b], accsp.at[dv[b]], ssem[b], add=True)
        return 0

    lax.fori_loop(0, NCHUNK // NB, group, 0)
    for b in range(NB):
        wait_rows_sem(ssem, b)
    plsc.subcore_barrier()
    pltpu.sync_copy(accsp.at[pl.ds(r0, RPT)],
                    out_hbm.at[c, pl.ds(r0, RPT)])


_prop_call = functools.partial(
    pl.kernel,
    out_type=jax.ShapeDtypeStruct((NC, NPAD, HD), jnp.float32),
    mesh=_mesh,
    scratch_types=(
        [pltpu.VMEM((CH,), jnp.int32) for _ in range(NB)]
        + [pltpu.VMEM((CH,), jnp.int32) for _ in range(NB)]
        + [pltpu.VMEM((CH,), jnp.float32) for _ in range(NB)]
        + [pltpu.VMEM((CH, HD), jnp.float32) for _ in range(NB)]
        + [pltpu.SemaphoreType.DMA for _ in range(3 * NB)]
        + [pltpu.VMEM_SHARED((NPAD, HD), jnp.float32)]
    ),
    compiler_params=pltpu.CompilerParams(use_tc_tiling_on_sc=False),
)(_prop_body)


def _tc1_body(y_ref, w1_ref, degp_ref, xs2_ref, dinv_ref):
    xw = jnp.dot(y_ref[...], w1_ref[...], preferred_element_type=jnp.float32)
    deg = degp_ref[0, :N, :] + degp_ref[1, :N, :] + 1.0
    dinv = jnp.where(deg > 0, lax.rsqrt(deg), 0.0)
    xs = xw * dinv
    zpad = jnp.zeros((NPAD - N, HD), jnp.float32)
    xs2_ref[0, :N, :] = xs[:, :HD]
    xs2_ref[0, pl.ds(N, NPAD - N), :] = zpad
    xs2_ref[1, :N, :] = xs[:, HD:]
    xs2_ref[1, pl.ds(N, NPAD - N), :] = zpad
    dinv_ref[...] = dinv


def _tc1(Y, W1, degp3):
    return pl.pallas_call(
        _tc1_body,
        out_shape=[
            jax.ShapeDtypeStruct((NC, NPAD, HD), jnp.float32),
            jax.ShapeDtypeStruct((N, 1), jnp.float32),
        ],
    )(Y, W1, degp3)


def _tc2_body(acc_ref, dinv_ref, b1_ref, hs2_ref):
    dinv = dinv_ref[...]
    zpad = jnp.zeros((NPAD - N, HD), jnp.float32)
    for h in range(NC):
        s1 = acc_ref[h, :N, :]
        hh = jax.nn.relu(s1 * dinv + b1_ref[pl.ds(h * HD, HD)])
        hs2_ref[h, :N, :] = hh * dinv
        hs2_ref[h, pl.ds(N, NPAD - N), :] = zpad


def _tc2(acc, dinv, b1):
    return pl.pallas_call(
        _tc2_body,
        out_shape=jax.ShapeDtypeStruct((NC, NPAD, HD), jnp.float32),
    )(acc, dinv, b1)


def _tc3_body(acc_ref, dinv_ref, wmu_ref, bmu_ref, wlv_ref, blv_ref,
              mu_ref, lv_ref):
    dinv = dinv_ref[...]
    p_lo = acc_ref[0, :N, :] * dinv
    p_hi = acc_ref[1, :N, :] * dinv
    mu_ref[...] = (
        jnp.dot(p_lo, wmu_ref[:HD, :], preferred_element_type=jnp.float32)
        + jnp.dot(p_hi, wmu_ref[pl.ds(HD, HD), :],
                  preferred_element_type=jnp.float32)
        + bmu_ref[...])
    lv_ref[...] = (
        jnp.dot(p_lo, wlv_ref[:HD, :], preferred_element_type=jnp.float32)
        + jnp.dot(p_hi, wlv_ref[pl.ds(HD, HD), :],
                  preferred_element_type=jnp.float32)
        + blv_ref[...])


def _tc3(acc, dinv, W_mu, b_mu, W_lv, b_lv):
    lat = W_mu.shape[1]
    return pl.pallas_call(
        _tc3_body,
        out_shape=[
            jax.ShapeDtypeStruct((N, lat), jnp.float32),
            jax.ShapeDtypeStruct((N, lat), jnp.float32),
        ],
    )(acc, dinv, W_mu, b_mu, W_lv, b_lv)


@jax.jit
def kernel(Y, edge_index, edge_weight, W1, b1, W_mu, b_mu, W_lv, b_lv):
    src = edge_index[0].astype(jnp.int32)
    dst = edge_index[1].astype(jnp.int32)
    w = edge_weight.astype(jnp.float32)
    dst3 = dst.reshape(NW, NCHUNK_D, CH)
    w3 = w.reshape(NW, NCHUNK_D, CH)

    degp = _deg_call(dst3, w3)
    degp3 = degp.reshape(NC, NPAD, 1)
    xs2, dinv = _tc1(Y, W1, degp3)
    acc1 = _prop_call(xs2, src, dst, w)
    hs2 = _tc2(acc1, dinv, b1)
    acc2 = _prop_call(hs2, src, dst, w)
    mu, lv = _tc3(acc2, dinv, W_mu, b_mu, W_lv, b_lv)
    return (mu, lv)
